# baseline (speedup 1.0000x reference)
import functools

import jax
import jax.numpy as jnp
from jax import lax
from jax.experimental import pallas as pl
from jax.experimental.pallas import tpu as pltpu
from jax.experimental.pallas import tpu_sc as plsc

_ROWS = 32768
_COLS = 1024
_CW = 32
_BAND = 128
_NC = 2
_NS = 16
_NW = _NC * _NS
_RPW = _ROWS // _NW
_BR = 256
_NBLK = _RPW // _BR
_L = 16

_mesh = plsc.VectorSubcoreMesh(
    core_axis_name="c", subcore_axis_name="s", num_cores=_NC,
    num_subcores=_NS)


def _make_sc_pair(chunk_starts):
    @functools.partial(
        pl.kernel,
        out_type=[jax.ShapeDtypeStruct((_ROWS, _CW), jnp.float32)]
        * len(chunk_starts),
        mesh=_mesh,
        scratch_types=[
            pltpu.VMEM((_BR, _BAND), jnp.float32),
            pltpu.VMEM((_BR, _BAND), jnp.float32),
            pltpu.VMEM((_BR, _CW), jnp.float32),
            pltpu.VMEM((_BR, _CW), jnp.float32),
            pltpu.SemaphoreType.DMA,
            pltpu.SemaphoreType.DMA,
            pltpu.SemaphoreType.DMA,
            pltpu.SemaphoreType.DMA,
        ],
    )
    def _sc_pair(x_hbm, *refs):
        outs = refs[:len(chunk_starts)]
        b0, b1, c0, c1, l0, l1, s0, s1 = refs[len(chunk_starts):]
        bufs = (b0, b1)
        cbufs = (c0, c1)
        lsems = (l0, l1)
        ssems = (s0, s1)
        wid = lax.axis_index("s") * _NC + lax.axis_index("c")
        base = wid * _RPW
        niter = len(chunk_starts) * _NBLK

        def rows(i):
            return pl.ds(base + (i % _NBLK) * _BR, _BR)

        def load(i):
            return pltpu.make_async_copy(
                x_hbm.at[rows(i), pl.ds(chunk_starts[i // _NBLK], _BAND)],
                bufs[i % 2], lsems[i % 2])

        def store(i):
            return pltpu.make_async_copy(
                cbufs[i % 2], outs[i // _NBLK].at[rows(i), :],
                ssems[i % 2])

        def compact(i):
            src = bufs[i % 2]
            dst = cbufs[i % 2]

            def body(r, carry):
                dst[r, pl.ds(0, _L)] = src[r, pl.ds(0, _L)]
                dst[r, pl.ds(_L, _L)] = src[r, pl.ds(_L, _L)]
                return carry

            lax.fori_loop(0, _BR, body, 0, unroll=8)

        load(0).start()
        for i in range(niter):
            if i + 1 < niter:
                load(i + 1).start()
            load(i).wait()
            if i >= 2:
                store(i - 2).wait()
            compact(i)
            store(i).start()
        store(niter - 2).wait()
        store(niter - 1).wait()

    return _sc_pair


_sc_pair_a = _make_sc_pair((0, 256))

_TC_R = 8192


def _tc_body(x2_ref, x3_ref, o2_ref, o3_ref):
    o2_ref[...] = x2_ref[:, 0:_CW]
    o3_ref[...] = x3_ref[:, 0:_CW]


_tc_select = pl.pallas_call(
    _tc_body,
    grid=(_ROWS // _TC_R,),
    in_specs=[
        pl.BlockSpec((_TC_R, _BAND), lambda i, c=c: (i, c // _BAND))
        for c in (512, 768)
    ],
    out_specs=[pl.BlockSpec((_TC_R, _CW), lambda i: (i, 0))] * 2,
    out_shape=[jax.ShapeDtypeStruct((_ROWS, _CW), jnp.float32)] * 2,
)


def kernel(x):
    o0, o1 = _sc_pair_a(x)
    o2, o3 = _tc_select(x, x)
    return (o0, o1, o2, o3)


# speedup vs baseline: 1.0252x; 1.0252x over previous
import functools

import jax
import jax.numpy as jnp
from jax import lax
from jax.experimental import pallas as pl
from jax.experimental.pallas import tpu as pltpu
from jax.experimental.pallas import tpu_sc as plsc

_ROWS = 32768
_COLS = 1024
_CW = 32
_BAND = 128
_NC = 2
_NS = 16
_NW = _NC * _NS
_RPW = _ROWS // _NW
_BR = 256
_NBLK = _RPW // _BR
_L = 16

_mesh = plsc.VectorSubcoreMesh(
    core_axis_name="c", subcore_axis_name="s", num_cores=_NC,
    num_subcores=_NS)


def _make_sc_pair(chunk_starts):
    @functools.partial(
        pl.kernel,
        out_type=[jax.ShapeDtypeStruct((_ROWS, _CW), jnp.float32)]
        * len(chunk_starts),
        mesh=_mesh,
        scratch_types=[
            pltpu.VMEM((_BR, _BAND), jnp.float32),
            pltpu.VMEM((_BR, _BAND), jnp.float32),
            pltpu.VMEM((_BR, _CW), jnp.float32),
            pltpu.VMEM((_BR, _CW), jnp.float32),
            pltpu.SemaphoreType.DMA,
            pltpu.SemaphoreType.DMA,
            pltpu.SemaphoreType.DMA,
            pltpu.SemaphoreType.DMA,
        ],
    )
    def _sc_pair(x_hbm, *refs):
        outs = refs[:len(chunk_starts)]
        b0, b1, c0, c1, l0, l1, s0, s1 = refs[len(chunk_starts):]
        bufs = (b0, b1)
        cbufs = (c0, c1)
        lsems = (l0, l1)
        ssems = (s0, s1)
        wid = lax.axis_index("s") * _NC + lax.axis_index("c")
        base = wid * _RPW
        niter = len(chunk_starts) * _NBLK

        def rows(i):
            return pl.ds(base + (i % _NBLK) * _BR, _BR)

        def load(i):
            return pltpu.make_async_copy(
                x_hbm.at[rows(i), pl.ds(chunk_starts[i // _NBLK], _BAND)],
                bufs[i % 2], lsems[i % 2])

        def store(i):
            return pltpu.make_async_copy(
                cbufs[i % 2], outs[i // _NBLK].at[rows(i), :],
                ssems[i % 2])

        def compact(i):
            src = bufs[i % 2]
            dst = cbufs[i % 2]

            def body(r, carry):
                dst[r, pl.ds(0, _L)] = src[r, pl.ds(0, _L)]
                dst[r, pl.ds(_L, _L)] = src[r, pl.ds(_L, _L)]
                return carry

            lax.fori_loop(0, _BR, body, 0, unroll=8)

        load(0).start()
        for i in range(niter):
            if i + 1 < niter:
                load(i + 1).start()
            load(i).wait()
            if i >= 2:
                store(i - 2).wait()
            compact(i)
            store(i).start()
        store(niter - 2).wait()
        store(niter - 1).wait()

    return _sc_pair


_sc_pair_a = _make_sc_pair((0, 256))
_sc_single_b = _make_sc_pair((512,))

_TC_R = 32768


def _tc_body(x3_ref, o3_ref):
    o3_ref[...] = x3_ref[:, 0:_CW]


_tc_select = pl.pallas_call(
    _tc_body,
    grid=(_ROWS // _TC_R,),
    in_specs=[pl.BlockSpec((_TC_R, _BAND), lambda i: (i, 768 // _BAND))],
    out_specs=[pl.BlockSpec((_TC_R, _CW), lambda i: (i, 0))],
    out_shape=[jax.ShapeDtypeStruct((_ROWS, _CW), jnp.float32)],
)


def kernel(x):
    o0, o1 = _sc_pair_a(x)
    (o2,) = _sc_single_b(x)
    (o3,) = _tc_select(x)
    return (o0, o1, o2, o3)
